# Initial kernel scaffold; baseline (speedup 1.0000x reference)
import functools

import jax
import jax.numpy as jnp
from jax import lax
from jax.experimental import pallas as pl
from jax.experimental.pallas import tpu as pltpu
from jax.experimental.pallas import tpu_sc as plsc

VOCAB = 32000
D = 512
B = 1024
L = 200

NC = 2
NS = 16
NW = NC * NS
ROWS_PER_W = B // NW
CHUNK = 40
NCHUNK = L // CHUNK
NCH = D // 16

_mesh = plsc.VectorSubcoreMesh(core_axis_name="c", subcore_axis_name="s")


@functools.partial(
    pl.kernel,
    mesh=_mesh,
    out_type=jax.ShapeDtypeStruct((B, D), jnp.float32),
    scratch_types=[
        pltpu.VMEM((ROWS_PER_W, L), jnp.int32),
        pltpu.VMEM((CHUNK, D), jnp.float32),
        pltpu.VMEM((CHUNK, D), jnp.float32),
        pltpu.VMEM((ROWS_PER_W, D), jnp.float32),
        pltpu.SemaphoreType.DMA,
        pltpu.SemaphoreType.DMA,
    ],
)
def _pool(ids_hbm, table_hbm, out_hbm, ids_v, g0, g1, outb, sem0, sem1):
    wid = lax.axis_index("s") * NC + lax.axis_index("c")
    base = wid * ROWS_PER_W
    pltpu.sync_copy(ids_hbm.at[pl.ds(base, ROWS_PER_W)], ids_v)

    gbufs = (g0, g1)
    sems = (sem0, sem1)

    def row_body(r, carry):
        handles = [None, None]
        handles[0] = pltpu.async_copy(
            table_hbm.at[ids_v.at[r, pl.ds(0, CHUNK)]], g0, sem0)
        accs = tuple(jnp.zeros((16,), jnp.float32) for _ in range(NCH))
        for k in range(NCHUNK):
            if k + 1 < NCHUNK:
                handles[(k + 1) % 2] = pltpu.async_copy(
                    table_hbm.at[ids_v.at[r, pl.ds((k + 1) * CHUNK, CHUNK)]],
                    gbufs[(k + 1) % 2], sems[(k + 1) % 2])
            handles[k % 2].wait()
            g = gbufs[k % 2]

            def chunk_body(l, a):
                return tuple(a[i] + g[l, pl.ds(i * 16, 16)] for i in range(NCH))

            accs = lax.fori_loop(0, CHUNK, chunk_body, accs)
        inv = jnp.float32(1.0 / L)
        for i in range(NCH):
            outb[r, pl.ds(i * 16, 16)] = accs[i] * inv
        return carry

    lax.fori_loop(0, ROWS_PER_W, row_body, 0)
    pltpu.sync_copy(outb, out_hbm.at[pl.ds(base, ROWS_PER_W)])


BV = 1280


def _mm_body(p_ref, w_ref, b_ref, o_ref):
    acc = jax.lax.dot_general(
        p_ref[...], w_ref[...], (((1,), (1,)), ((), ())),
        preferred_element_type=jnp.float32)
    o_ref[...] = acc + b_ref[...][None, :]


def _project(pooled, W, b):
    return pl.pallas_call(
        _mm_body,
        grid=(VOCAB // BV,),
        in_specs=[
            pl.BlockSpec((B, D), lambda i: (0, 0)),
            pl.BlockSpec((BV, D), lambda i: (i, 0)),
            pl.BlockSpec((BV,), lambda i: (i,)),
        ],
        out_specs=pl.BlockSpec((B, BV), lambda i: (0, i)),
        out_shape=jax.ShapeDtypeStruct((B, VOCAB), jnp.float32),
    )(pooled, W, b)


def kernel(input_ids, emb_table, W, b):
    ids = input_ids.astype(jnp.int32)
    pooled = _pool(ids, emb_table)
    return _project(pooled, W, b)


# speedup vs baseline: 1.9874x; 1.9874x over previous
import functools

import jax
import jax.numpy as jnp
from jax import lax
from jax.experimental import pallas as pl
from jax.experimental.pallas import tpu as pltpu
from jax.experimental.pallas import tpu_sc as plsc

VOCAB = 32000
D = 512
B = 1024
L = 200

NC = 2
NS = 16
NW = NC * NS
ROWS_PER_W = B // NW
CHUNK = 40
NCHUNK = L // CHUNK
NCH = D // 16

_mesh = plsc.VectorSubcoreMesh(core_axis_name="c", subcore_axis_name="s")


@functools.partial(
    pl.kernel,
    mesh=_mesh,
    out_type=jax.ShapeDtypeStruct((B, D), jnp.float32),
    scratch_types=[
        pltpu.VMEM((ROWS_PER_W * L,), jnp.int32),
        pltpu.VMEM((CHUNK, D), jnp.float32),
        pltpu.VMEM((CHUNK, D), jnp.float32),
        pltpu.VMEM((ROWS_PER_W, D), jnp.float32),
        pltpu.SemaphoreType.DMA,
        pltpu.SemaphoreType.DMA,
    ],
)
def _pool(ids_hbm, table_hbm, out_hbm, ids_v, g0, g1, outb, sem0, sem1):
    wid = lax.axis_index("s") * NC + lax.axis_index("c")
    base = wid * ROWS_PER_W
    pltpu.sync_copy(ids_hbm.at[pl.ds(base * L, ROWS_PER_W * L)], ids_v)

    gbufs = (g0, g1)
    sems = (sem0, sem1)

    def row_body(r, carry):
        handles = [None, None]
        handles[0] = pltpu.async_copy(
            table_hbm.at[ids_v.at[pl.ds(r * L, CHUNK)]], g0, sem0)
        accs = tuple(jnp.zeros((16,), jnp.float32) for _ in range(NCH))
        for k in range(NCHUNK):
            if k + 1 < NCHUNK:
                handles[(k + 1) % 2] = pltpu.async_copy(
                    table_hbm.at[ids_v.at[pl.ds(r * L + (k + 1) * CHUNK, CHUNK)]],
                    gbufs[(k + 1) % 2], sems[(k + 1) % 2])
            handles[k % 2].wait()
            g = gbufs[k % 2]

            def chunk_body(l, a):
                return tuple(a[i] + g[l, pl.ds(i * 16, 16)] for i in range(NCH))

            accs = lax.fori_loop(0, CHUNK, chunk_body, accs)
        inv = jnp.float32(1.0 / L)
        for i in range(NCH):
            outb[r, pl.ds(i * 16, 16)] = accs[i] * inv
        return carry

    lax.fori_loop(0, ROWS_PER_W, row_body, 0)
    pltpu.sync_copy(outb, out_hbm.at[pl.ds(base, ROWS_PER_W)])


BV = 1280


def _mm_body(p_ref, w_ref, b_ref, o_ref):
    acc = jax.lax.dot_general(
        p_ref[...], w_ref[...], (((1,), (1,)), ((), ())),
        preferred_element_type=jnp.float32)
    o_ref[...] = acc + b_ref[...]


def _project(pooled, W, b):
    return pl.pallas_call(
        _mm_body,
        grid=(VOCAB // BV,),
        in_specs=[
            pl.BlockSpec((B, D), lambda i: (0, 0)),
            pl.BlockSpec((BV, D), lambda i: (i, 0)),
            pl.BlockSpec((1, BV), lambda i: (0, i)),
        ],
        out_specs=pl.BlockSpec((B, BV), lambda i: (0, i)),
        out_shape=jax.ShapeDtypeStruct((B, VOCAB), jnp.float32),
    )(pooled, W, b.reshape(1, VOCAB))


def kernel(input_ids, emb_table, W, b):
    ids = input_ids.astype(jnp.int32).reshape(B * L)
    pooled = _pool(ids, emb_table)
    return _project(pooled, W, b)
